# baseline (speedup 1.0000x reference)
import functools

import jax
import jax.numpy as jnp
from jax import lax
from jax.experimental import pallas as pl
from jax.experimental.pallas import tpu as pltpu
from jax.experimental.pallas import tpu_sc as plsc

GRID_N = 16
NV = GRID_N * GRID_N
D = 768
B = 65536
NC, NS = 2, 16
NW = NC * NS
PER_W = B // NW
CHUNK = 64
NCH = PER_W // CHUNK
ROWS_W = NV // NW

_MESH = plsc.VectorSubcoreMesh(core_axis_name="c", subcore_axis_name="s")


@functools.partial(
    pl.kernel,
    mesh=_MESH,
    out_type=jax.ShapeDtypeStruct((NV, D), jnp.float32),
    scratch_types=[
        pltpu.VMEM((16, D), jnp.float32),
        pltpu.VMEM((16, D), jnp.float32),
        pltpu.VMEM((16,), jnp.int32),
        pltpu.VMEM((16,), jnp.int32),
        pltpu.SemaphoreType.DMA,
        pltpu.SemaphoreType.DMA,
    ],
)
def _build_comb(crd0_hbm, crd1_hbm, emb0_hbm, emb1_hbm, comb_hbm,
                buf0, buf1, crd0_v, crd1_v, s0, s1):
    cid = lax.axis_index("c")
    sid = lax.axis_index("s")
    wid = sid * NC + cid
    base = wid * ROWS_W
    pltpu.sync_copy(crd0_hbm.at[pl.ds(base, 16)], crd0_v)
    pltpu.sync_copy(crd1_hbm.at[pl.ds(base, 16)], crd1_v)
    c0 = crd0_v[...]
    c1 = crd1_v[...]
    cp0 = pltpu.async_copy(emb0_hbm.at[c0], buf0, s0)
    cp1 = pltpu.async_copy(emb1_hbm.at[c1], buf1, s1)
    cp0.wait()
    cp1.wait()

    def addrow(r, carry):
        for f in range(D // 16):
            sl = pl.ds(f * 16, 16)
            buf0[r, sl] = buf0[r, sl] + buf1[r, sl]
        return carry

    lax.fori_loop(0, ROWS_W, addrow, 0)
    pltpu.sync_copy(buf0.at[pl.ds(0, ROWS_W)], comb_hbm.at[pl.ds(base, ROWS_W)])


NBUF = 4
CH3 = 32
NCH3 = PER_W // CH3


@functools.partial(
    pl.kernel,
    mesh=_MESH,
    out_type=jax.ShapeDtypeStruct((B, D), jnp.float32),
    scratch_types=[
        pltpu.VMEM_SHARED((NV, D), jnp.float32),
        pltpu.VMEM((PER_W,), jnp.int32),
        pltpu.VMEM((NCH3, CH3), jnp.int32),
        pltpu.VMEM((NBUF, CH3, D), jnp.float32),
        pltpu.SemaphoreType.DMA,
        pltpu.SemaphoreType.DMA,
        pltpu.SemaphoreType.DMA,
        pltpu.SemaphoreType.DMA,
        pltpu.SemaphoreType.DMA,
        pltpu.SemaphoreType.DMA,
        pltpu.SemaphoreType.DMA,
        pltpu.SemaphoreType.DMA,
    ],
)
def _gather(pos_hbm, pos2_hbm, comb_hbm, out_hbm,
            table_sh, idx_v, idx2_v, bufs,
            sw0, sw1, sw2, sw3, sf0, sf1, sf2, sf3):
    cid = lax.axis_index("c")
    sid = lax.axis_index("s")
    wid = sid * NC + cid
    row0 = wid * PER_W

    t0 = sid * GRID_N
    pltpu.sync_copy(comb_hbm.at[pl.ds(t0, GRID_N)], table_sh.at[pl.ds(t0, GRID_N)])
    pltpu.sync_copy(pos_hbm.at[pl.ds(row0, PER_W)], idx_v)
    pltpu.sync_copy(pos2_hbm.at[pl.ds(wid * NCH3, NCH3)], idx2_v)
    plsc.subcore_barrier()

    sw = (sw0, sw1, sw2, sw3)
    sf = (sf0, sf1, sf2, sf3)

    def wstart(g, b):
        pltpu.async_copy(bufs.at[b],
                         out_hbm.at[pl.ds(row0 + g * CH3, CH3)], sw[b])

    def wwait(b):
        pltpu.make_async_copy(bufs.at[b],
                              out_hbm.at[pl.ds(row0, CH3)], sw[b]).wait()

    def fill_issue(cbase, b):
        def qbody(q, carry):
            pvec = idx_v[pl.ds(cbase + q * 16, 16)]
            for j in range(16):
                p = pvec[j]
                pltpu.async_copy(table_sh.at[pl.ds(p, 1)],
                                 bufs.at[b, pl.ds(q * 16 + j, 1)], sf[b])
            return carry
        lax.fori_loop(0, CH3 // 16, qbody, 0)

    def fill_issue_hbm(g, b):
        pltpu.async_copy(comb_hbm.at[idx2_v.at[g]], bufs.at[b], sf[b])

    def issue_any(g, b):
        @pl.when(g % 8 == 0)
        def _():
            fill_issue_hbm(g, b)

        @pl.when(g % 8 != 0)
        def _():
            fill_issue(g * CH3, b)

    def fill_drain(b):
        pltpu.make_async_copy(table_sh.at[pl.ds(0, CH3)],
                              bufs.at[b], sf[b]).wait()

    fill_issue_hbm(0, 0)

    def body(k, carry):
        for par in range(NBUF):
            g = NBUF * k + par

            @pl.when(g + 1 < NCH3)
            def _():
                nb = (par + 1) % NBUF

                @pl.when(g + 1 >= NBUF)
                def _():
                    wwait(nb)

                issue_any(g + 1, nb)

            fill_drain(par)
            wstart(g, par)
        return carry

    lax.fori_loop(0, NCH3 // NBUF, body, 0)

    for b in range(NBUF):
        wwait(b)


def kernel(positions, coords, emb0, emb1):
    pos = positions.astype(jnp.int32)
    crd = coords.astype(jnp.int32)
    pad = jnp.zeros((16,), jnp.int32)
    crd0 = jnp.concatenate([crd[:, 0], pad])
    crd1 = jnp.concatenate([crd[:, 1], pad])
    comb = _build_comb(crd0, crd1, emb0.astype(jnp.float32),
                       emb1.astype(jnp.float32))
    return _gather(pos, pos.reshape(B // CH3, CH3), comb)


# speedup vs baseline: 1.0549x; 1.0549x over previous
import functools

import jax
import jax.numpy as jnp
from jax import lax
from jax.experimental import pallas as pl
from jax.experimental.pallas import tpu as pltpu
from jax.experimental.pallas import tpu_sc as plsc

GRID_N = 16
NV = GRID_N * GRID_N
D = 768
B = 65536
NC, NS = 2, 16
NW = NC * NS
PER_W = B // NW
CHUNK = 64
NCH = PER_W // CHUNK
ROWS_W = NV // NW

_MESH = plsc.VectorSubcoreMesh(core_axis_name="c", subcore_axis_name="s")


@functools.partial(
    pl.kernel,
    mesh=_MESH,
    out_type=jax.ShapeDtypeStruct((NV, D), jnp.float32),
    scratch_types=[
        pltpu.VMEM((16, D), jnp.float32),
        pltpu.VMEM((16, D), jnp.float32),
        pltpu.VMEM((16,), jnp.int32),
        pltpu.VMEM((16,), jnp.int32),
        pltpu.SemaphoreType.DMA,
        pltpu.SemaphoreType.DMA,
    ],
)
def _build_comb(crd0_hbm, crd1_hbm, emb0_hbm, emb1_hbm, comb_hbm,
                buf0, buf1, crd0_v, crd1_v, s0, s1):
    cid = lax.axis_index("c")
    sid = lax.axis_index("s")
    wid = sid * NC + cid
    base = wid * ROWS_W
    pltpu.sync_copy(crd0_hbm.at[pl.ds(base, 16)], crd0_v)
    pltpu.sync_copy(crd1_hbm.at[pl.ds(base, 16)], crd1_v)
    c0 = crd0_v[...]
    c1 = crd1_v[...]
    cp0 = pltpu.async_copy(emb0_hbm.at[c0], buf0, s0)
    cp1 = pltpu.async_copy(emb1_hbm.at[c1], buf1, s1)
    cp0.wait()
    cp1.wait()

    def addrow(r, carry):
        for f in range(D // 16):
            sl = pl.ds(f * 16, 16)
            buf0[r, sl] = buf0[r, sl] + buf1[r, sl]
        return carry

    lax.fori_loop(0, ROWS_W, addrow, 0)
    pltpu.sync_copy(buf0.at[pl.ds(0, ROWS_W)], comb_hbm.at[pl.ds(base, ROWS_W)])


NBUF = 4
CH3 = 32
NCH3 = PER_W // CH3


@functools.partial(
    pl.kernel,
    mesh=_MESH,
    out_type=jax.ShapeDtypeStruct((B, D), jnp.float32),
    scratch_types=[
        pltpu.VMEM_SHARED((NV, D), jnp.float32),
        pltpu.VMEM((PER_W,), jnp.int32),
        pltpu.VMEM((NBUF, CH3, D), jnp.float32),
        pltpu.SemaphoreType.DMA,
        pltpu.SemaphoreType.DMA,
        pltpu.SemaphoreType.DMA,
        pltpu.SemaphoreType.DMA,
        pltpu.SemaphoreType.DMA,
        pltpu.SemaphoreType.DMA,
        pltpu.SemaphoreType.DMA,
        pltpu.SemaphoreType.DMA,
    ],
)
def _gather(pos_hbm, comb_hbm, out_hbm,
            table_sh, idx_v, bufs,
            sw0, sw1, sw2, sw3, sf0, sf1, sf2, sf3):
    cid = lax.axis_index("c")
    sid = lax.axis_index("s")
    wid = sid * NC + cid
    row0 = wid * PER_W

    t0 = sid * GRID_N
    pltpu.sync_copy(comb_hbm.at[pl.ds(t0, GRID_N)], table_sh.at[pl.ds(t0, GRID_N)])
    pltpu.sync_copy(pos_hbm.at[pl.ds(row0, PER_W)], idx_v)
    plsc.subcore_barrier()

    sw = (sw0, sw1, sw2, sw3)
    sf = (sf0, sf1, sf2, sf3)

    def wstart(g, b):
        pltpu.async_copy(bufs.at[b],
                         out_hbm.at[pl.ds(row0 + g * CH3, CH3)], sw[b])

    def wwait(b):
        pltpu.make_async_copy(bufs.at[b],
                              out_hbm.at[pl.ds(row0, CH3)], sw[b]).wait()

    def fill_issue(cbase, b):
        def qbody(q, carry):
            pvec = idx_v[pl.ds(cbase + q * 16, 16)]
            for j in range(16):
                p = pvec[j]
                pltpu.async_copy(table_sh.at[pl.ds(p, 1)],
                                 bufs.at[b, pl.ds(q * 16 + j, 1)], sf[b])
            return carry
        lax.fori_loop(0, CH3 // 16, qbody, 0)

    def fill_drain(b):
        pltpu.make_async_copy(table_sh.at[pl.ds(0, CH3)],
                              bufs.at[b], sf[b]).wait()

    fill_issue(0, 0)
    fill_issue(CH3, 1)

    def body(k, carry):
        for par in range(NBUF):
            g = NBUF * k + par

            @pl.when(g + 2 < NCH3)
            def _():
                nb = (par + 2) % NBUF

                @pl.when(g + 2 >= NBUF)
                def _():
                    wwait(nb)

                fill_issue((g + 2) * CH3, nb)

            fill_drain(par)
            wstart(g, par)
        return carry

    lax.fori_loop(0, NCH3 // NBUF, body, 0)

    for b in range(NBUF):
        wwait(b)


def kernel(positions, coords, emb0, emb1):
    pos = positions.astype(jnp.int32)
    crd = coords.astype(jnp.int32)
    pad = jnp.zeros((16,), jnp.int32)
    crd0 = jnp.concatenate([crd[:, 0], pad])
    crd1 = jnp.concatenate([crd[:, 1], pad])
    comb = _build_comb(crd0, crd1, emb0.astype(jnp.float32),
                       emb1.astype(jnp.float32))
    return _gather(pos, comb)


# speedup vs baseline: 1.1119x; 1.0540x over previous
import functools

import jax
import jax.numpy as jnp
from jax import lax
from jax.experimental import pallas as pl
from jax.experimental.pallas import tpu as pltpu
from jax.experimental.pallas import tpu_sc as plsc

GRID_N = 16
NV = GRID_N * GRID_N
D = 768
B = 65536
NC, NS = 2, 16
NW = NC * NS
PER_W = B // NW
NBUF = 4
CH3 = 32
NCH3 = PER_W // CH3

_MESH = plsc.VectorSubcoreMesh(core_axis_name="c", subcore_axis_name="s")


@functools.partial(
    pl.kernel,
    mesh=_MESH,
    out_type=jax.ShapeDtypeStruct((B, D), jnp.float32),
    scratch_types=[
        pltpu.VMEM_SHARED((NV, D), jnp.float32),
        pltpu.VMEM((PER_W,), jnp.int32),
        pltpu.VMEM((NBUF, CH3, D), jnp.float32),
        pltpu.VMEM((GRID_N,), jnp.int32),
        pltpu.VMEM((GRID_N,), jnp.int32),
        pltpu.SemaphoreType.DMA,
        pltpu.SemaphoreType.DMA,
        pltpu.SemaphoreType.DMA,
        pltpu.SemaphoreType.DMA,
        pltpu.SemaphoreType.DMA,
        pltpu.SemaphoreType.DMA,
        pltpu.SemaphoreType.DMA,
        pltpu.SemaphoreType.DMA,
    ],
)
def _embed(pos_hbm, crd0_hbm, crd1_hbm, emb0_hbm, emb1_hbm, out_hbm,
           table_sh, idx_v, bufs, crd0_v, crd1_v,
           sw0, sw1, sw2, sw3, sf0, sf1, sf2, sf3):
    cid = lax.axis_index("c")
    sid = lax.axis_index("s")
    wid = sid * NC + cid
    row0 = wid * PER_W
    sw = (sw0, sw1, sw2, sw3)
    sf = (sf0, sf1, sf2, sf3)

    t0 = sid * GRID_N
    pltpu.sync_copy(crd0_hbm.at[pl.ds(t0, GRID_N)], crd0_v)
    pltpu.sync_copy(crd1_hbm.at[pl.ds(t0, GRID_N)], crd1_v)
    c0 = crd0_v[...]
    c1 = crd1_v[...]
    cp0 = pltpu.async_copy(emb0_hbm.at[c0], bufs.at[0, pl.ds(0, GRID_N)], sf0)
    cp1 = pltpu.async_copy(emb1_hbm.at[c1], bufs.at[1, pl.ds(0, GRID_N)], sf1)
    pltpu.sync_copy(pos_hbm.at[pl.ds(row0, PER_W)], idx_v)
    cp0.wait()
    cp1.wait()

    def addrow(r, carry):
        for f in range(D // 16):
            sl = pl.ds(f * 16, 16)
            bufs[0, r, sl] = bufs[0, r, sl] + bufs[1, r, sl]
        return carry

    lax.fori_loop(0, GRID_N, addrow, 0)
    pltpu.sync_copy(bufs.at[0, pl.ds(0, GRID_N)], table_sh.at[pl.ds(t0, GRID_N)])
    plsc.subcore_barrier()

    def wstart(g, b):
        pltpu.async_copy(bufs.at[b],
                         out_hbm.at[pl.ds(row0 + g * CH3, CH3)], sw[b])

    def wwait(b):
        pltpu.make_async_copy(bufs.at[b],
                              out_hbm.at[pl.ds(row0, CH3)], sw[b]).wait()

    def fill_issue(cbase, b):
        def qbody(q, carry):
            pvec = idx_v[pl.ds(cbase + q * 16, 16)]
            for j in range(16):
                p = pvec[j]
                pltpu.async_copy(table_sh.at[pl.ds(p, 1)],
                                 bufs.at[b, pl.ds(q * 16 + j, 1)], sf[b])
            return carry
        lax.fori_loop(0, CH3 // 16, qbody, 0)

    def fill_drain(b):
        pltpu.make_async_copy(table_sh.at[pl.ds(0, CH3)],
                              bufs.at[b], sf[b]).wait()

    fill_issue(0, 0)

    def body(k, carry):
        for par in range(NBUF):
            g = NBUF * k + par

            @pl.when(g + 1 < NCH3)
            def _():
                nb = (par + 1) % NBUF

                @pl.when(g + 1 >= NBUF)
                def _():
                    wwait(nb)

                fill_issue((g + 1) * CH3, nb)

            fill_drain(par)
            wstart(g, par)
        return carry

    lax.fori_loop(0, NCH3 // NBUF, body, 0)

    for b in range(NBUF):
        wwait(b)


def kernel(positions, coords, emb0, emb1):
    pos = positions.astype(jnp.int32)
    crd = coords.astype(jnp.int32)
    return _embed(pos, crd[:, 0].reshape(-1), crd[:, 1].reshape(-1),
                  emb0.astype(jnp.float32), emb1.astype(jnp.float32))
